# baseline (speedup 1.0000x reference)
import functools

import jax
import jax.numpy as jnp
from jax.experimental import pallas as pl
from jax.experimental.pallas import tpu as pltpu

_THRESHOLD = 0.0
_C = 192
_H = 512
_W = 512
_BLK_H = 64


def _estimator_kernel(x_ref, out_ref, stats_ref, bbox_ref):
    i = pl.program_id(0)
    n = pl.num_programs(0)

    @pl.when(i == 0)
    def _init():
        bbox_ref[0] = _H
        bbox_ref[1] = -1
        bbox_ref[2] = _W
        bbox_ref[3] = -1

    sums = jnp.sum(x_ref[...], axis=0)
    mask = sums >= _THRESHOLD

    row_idx = jax.lax.broadcasted_iota(jnp.int32, (_BLK_H, _W), 0) + i * _BLK_H
    col_idx = jax.lax.broadcasted_iota(jnp.int32, (_BLK_H, _W), 1)

    bbox_ref[0] = jnp.minimum(bbox_ref[0], jnp.min(jnp.where(mask, row_idx, _H)))
    bbox_ref[1] = jnp.maximum(bbox_ref[1], jnp.max(jnp.where(mask, row_idx, -1)))
    bbox_ref[2] = jnp.minimum(bbox_ref[2], jnp.min(jnp.where(mask, col_idx, _W)))
    bbox_ref[3] = jnp.maximum(bbox_ref[3], jnp.max(jnp.where(mask, col_idx, -1)))

    out_ref[...] = jnp.zeros_like(out_ref)

    @pl.when(i == n - 1)
    def _fin():
        y1 = bbox_ref[0]
        y2 = bbox_ref[1] + 1
        x1 = bbox_ref[2]
        x2 = bbox_ref[3] + 1
        has_any = y2 > 0
        frac = jnp.where(
            has_any,
            ((y2 - y1) * (x2 - x1)).astype(jnp.float32) / float(_H * _W),
            0.0,
        )
        stats_ref[...] = jnp.full((8, 128), frac, dtype=jnp.float32)


@functools.partial(jax.jit, static_argnames=("interpret",))
def kernel(x, interpret=False):
    xr = x.reshape(_C, _H, _W)
    out, _stats = pl.pallas_call(
        _estimator_kernel,
        grid=(_H // _BLK_H,),
        in_specs=[pl.BlockSpec((_C, _BLK_H, _W), lambda i: (0, i, 0))],
        out_specs=[
            pl.BlockSpec((_BLK_H, _W), lambda i: (i, 0)),
            pl.BlockSpec((8, 128), lambda i: (0, 0)),
        ],
        out_shape=[
            jax.ShapeDtypeStruct((_H, _W), x.dtype),
            jax.ShapeDtypeStruct((8, 128), jnp.float32),
        ],
        scratch_shapes=[pltpu.SMEM((4,), jnp.int32)],
        interpret=interpret,
    )(xr)
    return out.reshape(1, 1, _H, _W)
